# Initial kernel scaffold; baseline (speedup 1.0000x reference)
import jax
import jax.numpy as jnp
from jax.experimental import pallas as pl


def kernel(feat, select_idx, scores):
    raise NotImplementedError("write your pallas kernel here")


# speedup vs baseline: 1.2942x; 1.2942x over previous
import functools

import jax
import jax.numpy as jnp
from jax import lax
from jax.experimental import pallas as pl
from jax.experimental.pallas import tpu as pltpu
from jax.experimental.pallas import tpu_sc as plsc

NC = 2
NS = 16
NW = NC * NS
LANES = 16


def _make_kernel(N, D, K):
    align = 8 * NW
    KP = ((K + align - 1) // align) * align
    BPW = KP // NW
    C = 224
    while BPW % C != 0:
        C -= 16
    NCHUNK = BPW // C
    TAIL = K - ((NW - 1) * BPW + (NCHUNK - 1) * C)
    n_vec = D // LANES

    mesh = plsc.VectorSubcoreMesh(
        core_axis_name="c", subcore_axis_name="s",
        num_cores=NC, num_subcores=NS)

    @functools.partial(
        pl.kernel,
        out_type=jax.ShapeDtypeStruct((K, D), jnp.float32),
        mesh=mesh,
        scratch_types=[
            pltpu.VMEM((C,), jnp.int32),
            pltpu.VMEM((C,), jnp.float32),
            pltpu.VMEM((C, D), jnp.float32),
            pltpu.SemaphoreType.DMA,
        ],
    )
    def gather_scale(feat_hbm, idx_hbm, scores_hbm, out_hbm,
                     idx_v, sc_v, rows_v, sem):
        cid = lax.axis_index("c")
        sid = lax.axis_index("s")
        wid = sid * NC + cid
        base = wid * BPW

        def scale_rows(g, _):
            sv = sc_v[pl.ds(g * LANES, LANES)]
            for i in range(LANES):
                r = g * LANES + i
                s = sv[i]
                for j in range(n_vec):
                    sl = pl.ds(j * LANES, LANES)
                    rows_v[r, sl] = rows_v[r, sl] * s
            return _

        for k in range(NCHUNK):
            off = base + k * C
            pltpu.sync_copy(idx_hbm.at[pl.ds(off, C)], idx_v)
            pltpu.sync_copy(scores_hbm.at[pl.ds(off, C)], sc_v)
            pltpu.async_copy(feat_hbm.at[idx_v], rows_v, sem).wait()
            lax.fori_loop(0, C // LANES, scale_rows, None)
            if k < NCHUNK - 1 or TAIL == C:
                pltpu.sync_copy(rows_v, out_hbm.at[pl.ds(off, C)])
            else:
                @pl.when(wid < NW - 1)
                def _():
                    pltpu.sync_copy(rows_v, out_hbm.at[pl.ds(off, C)])

                @pl.when(wid == NW - 1)
                def _():
                    pltpu.sync_copy(rows_v.at[pl.ds(0, TAIL)],
                                    out_hbm.at[pl.ds(off, TAIL)])

    return gather_scale, KP


def kernel(feat, select_idx, scores):
    N, D = feat.shape
    K = select_idx.shape[0]
    fn, KP = _make_kernel(N, D, K)
    idx = select_idx.astype(jnp.int32)
    pad = KP - K
    if pad:
        idx = jnp.concatenate([idx, jnp.zeros((pad,), jnp.int32)])
        scores = jnp.concatenate([scores, jnp.zeros((pad,), jnp.float32)])
    return fn(feat, idx, scores)


# speedup vs baseline: 1.6071x; 1.2417x over previous
import functools

import jax
import jax.numpy as jnp
from jax import lax
from jax.experimental import pallas as pl
from jax.experimental.pallas import tpu as pltpu
from jax.experimental.pallas import tpu_sc as plsc

NC = 2
NS = 16
NW = NC * NS
LANES = 16


def _make_kernel(N, D, K):
    align = 8 * NW
    KP = ((K + align - 1) // align) * align
    BPW = KP // NW
    C = 224
    while BPW % C != 0:
        C -= 16
    NCHUNK = BPW // C
    TAIL = K - ((NW - 1) * BPW + (NCHUNK - 1) * C)
    n_vec = D // LANES

    mesh = plsc.VectorSubcoreMesh(
        core_axis_name="c", subcore_axis_name="s",
        num_cores=NC, num_subcores=NS)

    @functools.partial(
        pl.kernel,
        out_type=jax.ShapeDtypeStruct((K, D), jnp.float32),
        mesh=mesh,
        scratch_types=[
            pltpu.VMEM((BPW,), jnp.int32),
            pltpu.VMEM((BPW,), jnp.float32),
            pltpu.VMEM((2, C, D), jnp.float32),
            pltpu.SemaphoreType.DMA,
            pltpu.SemaphoreType.DMA,
            pltpu.SemaphoreType.DMA,
            pltpu.SemaphoreType.DMA,
        ],
    )
    def gather_scale(feat_hbm, idx_hbm, scores_hbm, out_hbm,
                     idx_v, sc_v, rows_v, g0, g1, o0, o1):
        cid = lax.axis_index("c")
        sid = lax.axis_index("s")
        wid = sid * NC + cid
        base = wid * BPW
        gsem = (g0, g1)
        osem = (o0, o1)

        pltpu.sync_copy(idx_hbm.at[pl.ds(base, BPW)], idx_v)
        pltpu.sync_copy(scores_hbm.at[pl.ds(base, BPW)], sc_v)

        def start_gather(k):
            b = k % 2
            return pltpu.async_copy(
                feat_hbm.at[idx_v.at[pl.ds(k * C, C)]],
                rows_v.at[b], gsem[b])

        def scale_chunk(k):
            b = k % 2

            def body(g, _):
                sv = sc_v[pl.ds(k * C + g * LANES, LANES)]
                for i in range(LANES):
                    s = sv[i]
                    for j in range(n_vec):
                        sl = pl.ds(j * LANES, LANES)
                        rows_v[b, g * LANES + i, sl] = \
                            rows_v[b, g * LANES + i, sl] * s
                return _

            lax.fori_loop(0, C // LANES, body, None)

        gdesc = {0: start_gather(0)}
        odesc = {}
        for k in range(NCHUNK):
            b = k % 2
            off = base + k * C
            if k + 1 < NCHUNK:
                if k - 1 in odesc:
                    odesc.pop(k - 1).wait()
                gdesc[k + 1] = start_gather(k + 1)
            gdesc.pop(k).wait()
            scale_chunk(k)
            if k < NCHUNK - 1 or TAIL == C:
                odesc[k] = pltpu.async_copy(
                    rows_v.at[b], out_hbm.at[pl.ds(off, C)], osem[b])
            else:
                @pl.when(wid < NW - 1)
                def _():
                    pltpu.async_copy(
                        rows_v.at[b], out_hbm.at[pl.ds(off, C)],
                        osem[b]).wait()

                @pl.when(wid == NW - 1)
                def _():
                    pltpu.async_copy(
                        rows_v.at[b].at[pl.ds(0, TAIL)],
                        out_hbm.at[pl.ds(off, TAIL)], osem[b]).wait()
        for k in sorted(odesc):
            odesc.pop(k).wait()

    return gather_scale, KP


def kernel(feat, select_idx, scores):
    N, D = feat.shape
    K = select_idx.shape[0]
    fn, KP = _make_kernel(N, D, K)
    idx = select_idx.astype(jnp.int32)
    pad = KP - K
    if pad:
        idx = jnp.concatenate([idx, jnp.zeros((pad,), jnp.int32)])
        scores = jnp.concatenate([scores, jnp.zeros((pad,), jnp.float32)])
    return fn(feat, idx, scores)
